# Initial kernel scaffold; baseline (speedup 1.0000x reference)
import jax
import jax.numpy as jnp
from jax.experimental import pallas as pl


def kernel(image, label, keypoints, mask, probe):
    raise NotImplementedError("write your pallas kernel here")


# speedup vs baseline: 30.6334x; 30.6334x over previous
import numpy as np
import jax
import jax.numpy as jnp
from jax.experimental import pallas as pl

_MIN_SALT, _MAX_SALT = 0.005, 0.01
_MIN_PEPPER, _MAX_PEPPER = 0.005, 0.01

_H = _W = 512

_nk = jax.random.key(1234)
_ka, _kb, _kc, _kd = jax.random.split(_nk, 4)
_u_salt = float(jax.random.uniform(_ka, ()))
_u_pepper = float(jax.random.uniform(_kb, ()))
_n_salt = int((_MIN_SALT + _u_salt * (_MAX_SALT - _MIN_SALT)) * _H * _W)
_n_pepper = int((_MIN_PEPPER + _u_pepper * (_MAX_PEPPER - _MIN_PEPPER)) * _H * _W)
_salt_locs = np.asarray(jax.random.randint(_kc, (_n_salt,), 0, _W * _H - 1))
_pepper_locs = np.asarray(jax.random.randint(_kd, (_n_pepper,), 0, _W * _H - 1))

_code_np = np.zeros((_H * _W,), np.uint8)
_code_np[_salt_locs] = 1
_code_np[_pepper_locs] = 2
_CODE = jnp.asarray(_code_np.reshape(_H, _W))

_ROWS = 64
_GRID = _H // _ROWS


def _body(img_ref, mask_ref, code_ref, out_ref):
    img = img_ref[...]
    m = mask_ref[...]
    code = code_ref[...][None]
    v = jnp.where(code == 1, 255.0, img)
    v = jnp.where(code == 2, 0.0, v)
    out_ref[...] = (v * m).astype(jnp.uint8)


def _noise(image, mask):
    return pl.pallas_call(
        _body,
        grid=(_GRID,),
        in_specs=[
            pl.BlockSpec((3, _ROWS, _W), lambda i: (0, i, 0)),
            pl.BlockSpec((1, _ROWS, _W), lambda i: (0, i, 0)),
            pl.BlockSpec((_ROWS, _W), lambda i: (i, 0)),
        ],
        out_specs=pl.BlockSpec((3, _ROWS, _W), lambda i: (0, i, 0)),
        out_shape=jax.ShapeDtypeStruct((3, _H, _W), jnp.uint8),
    )(image, mask, _CODE)


def kernel(image, label, keypoints, mask, probe):
    new_image = _noise(image, mask)
    return (new_image, label, keypoints, mask, probe)
